# Initial kernel scaffold; baseline (speedup 1.0000x reference)
import jax
import jax.numpy as jnp
from jax.experimental import pallas as pl


def kernel(a, b, table):
    raise NotImplementedError("write your pallas kernel here")


# speedup vs baseline: 181.4714x; 181.4714x over previous
import functools

import jax
import jax.numpy as jnp
from jax import lax
from jax.experimental import pallas as pl
from jax.experimental.pallas import tpu as pltpu
from jax.experimental.pallas import tpu_sc as plsc

L = 16
NC, NS = 2, 16
NW = NC * NS
ROWS, COLS = 16384, 200
N = ROWS * COLS
WORDS = N // 2
WPT = WORDS // NW
NCHUNK = 8
WCH = WPT // NCHUNK
ECH = 2 * WCH
TAB_WORDS = 256 * 128


@functools.partial(
    pl.kernel,
    out_type=jax.ShapeDtypeStruct((WORDS,), jnp.int32),
    mesh=plsc.VectorSubcoreMesh(core_axis_name="c", subcore_axis_name="s"),
    compiler_params=pltpu.CompilerParams(needs_layout_passes=False),
    scratch_types=[
        pltpu.VMEM((TAB_WORDS,), jnp.int32),
        pltpu.VMEM((256,), jnp.int32),
        pltpu.VMEM((L,), jnp.int32),
        pltpu.VMEM((ECH,), jnp.int32),
        pltpu.VMEM((ECH,), jnp.int32),
        pltpu.VMEM((WCH,), jnp.int32),
        pltpu.VMEM((WCH,), jnp.int32),
        pltpu.SemaphoreType.DMA,
        pltpu.SemaphoreType.DMA,
        pltpu.SemaphoreType.DMA,
        pltpu.SemaphoreType.DMA,
    ],
)
def _lut_kernel(tab_hbm, a_hbm, b_hbm, out_hbm,
                tab_v, col_v, b_v, a0_v, a1_v, o0_v, o1_v,
                is0, is1, os0, os1):
    wid = lax.axis_index("s") * NC + lax.axis_index("c")
    ebase = wid * (2 * WPT)
    obase = wid * WPT
    a_bufs = (a0_v, a1_v)
    o_bufs = (o0_v, o1_v)
    i_sems = (is0, is1)
    o_sems = (os0, os1)

    in_cp = {}
    for c in range(2):
        in_cp[c] = pltpu.async_copy(
            a_hbm.at[pl.ds(ebase + c * ECH, ECH)], a_bufs[c], i_sems[c])
    pltpu.sync_copy(b_hbm, b_v)
    pltpu.sync_copy(tab_hbm, tab_v)

    iota = lax.iota(jnp.int32, L)
    cb = b_v[...] + 128
    wcol = lax.shift_right_arithmetic(cb, 1)
    odd = lax.eq(lax.bitwise_and(cb, 1), 1)
    for j in range(256 // L):
        r = j * L + iota
        w = plsc.load_gather(tab_v, [r * 128 + wcol])
        lo = lax.shift_right_arithmetic(lax.shift_left(w, 16), 16)
        hi = lax.shift_right_arithmetic(w, 16)
        col_v[pl.ds(j * L, L)] = lax.select(odd, hi, lo)

    iota2 = iota * 2

    def compute_chunk(a_ref, o_ref):
        def body(g, carry):
            eb = g * (2 * L)
            av_e = plsc.load_gather(a_ref, [eb + iota2])
            av_o = plsc.load_gather(a_ref, [eb + iota2 + 1])
            ce = plsc.load_gather(col_v, [av_e + 128])
            co = plsc.load_gather(col_v, [av_o + 128])
            word = lax.bitwise_or(
                lax.bitwise_and(ce, 0xFFFF), lax.shift_left(co, 16))
            o_ref[pl.ds(g * L, L)] = word
            return carry
        lax.fori_loop(0, WCH // L, body, 0)

    out_cp = {}
    for c in range(NCHUNK):
        p = c & 1
        in_cp[c].wait()
        if c >= 2:
            out_cp[c - 2].wait()
        compute_chunk(a_bufs[p], o_bufs[p])
        out_cp[c] = pltpu.async_copy(
            o_bufs[p], out_hbm.at[pl.ds(obase + c * WCH, WCH)], o_sems[p])
        if c + 2 < NCHUNK:
            in_cp[c + 2] = pltpu.async_copy(
                a_hbm.at[pl.ds(ebase + (c + 2) * ECH, ECH)], a_bufs[p],
                i_sems[p])
    out_cp[NCHUNK - 2].wait()
    out_cp[NCHUNK - 1].wait()


def kernel(a, b, table):
    tab_w = lax.bitcast_convert_type(
        table.reshape(TAB_WORDS, 2), jnp.int32)
    a_flat = a.astype(jnp.int32).reshape(N)
    b_vec = jnp.full((L,), b, dtype=jnp.int32)
    out_w = _lut_kernel(tab_w, a_flat, b_vec)
    return lax.bitcast_convert_type(out_w, jnp.int16).reshape(ROWS, COLS)


# speedup vs baseline: 216.4858x; 1.1929x over previous
import functools

import jax
import jax.numpy as jnp
from jax import lax
from jax.experimental import pallas as pl
from jax.experimental.pallas import tpu as pltpu
from jax.experimental.pallas import tpu_sc as plsc

L = 16
NC, NS = 2, 16
NW = NC * NS
ROWS, COLS = 16384, 200
N = ROWS * COLS
WORDS = N // 2
WPT = WORDS // NW
NCHUNK = 8
WCH = WPT // NCHUNK
ECH = 2 * WCH
TAB_WORDS = 256 * 128


@functools.partial(
    pl.kernel,
    out_type=jax.ShapeDtypeStruct((WORDS,), jnp.int32),
    mesh=plsc.VectorSubcoreMesh(core_axis_name="c", subcore_axis_name="s"),
    compiler_params=pltpu.CompilerParams(needs_layout_passes=False),
    scratch_types=[
        pltpu.VMEM((TAB_WORDS,), jnp.int32),
        pltpu.VMEM((256,), jnp.int32),
        pltpu.VMEM((L,), jnp.int32),
        pltpu.VMEM((ECH,), jnp.int32),
        pltpu.VMEM((ECH,), jnp.int32),
        pltpu.VMEM((WCH,), jnp.int32),
        pltpu.VMEM((WCH,), jnp.int32),
        pltpu.SemaphoreType.DMA,
        pltpu.SemaphoreType.DMA,
        pltpu.SemaphoreType.DMA,
        pltpu.SemaphoreType.DMA,
    ],
)
def _lut_kernel(tab_hbm, a_hbm, b_hbm, out_hbm,
                tab_v, col_v, b_v, a0_v, a1_v, o0_v, o1_v,
                is0, is1, os0, os1):
    wid = lax.axis_index("s") * NC + lax.axis_index("c")
    ebase = wid * (2 * WPT)
    obase = wid * WPT
    a_bufs = (a0_v, a1_v)
    o_bufs = (o0_v, o1_v)
    i_sems = (is0, is1)
    o_sems = (os0, os1)

    in_cp = {}
    for c in range(2):
        in_cp[c] = pltpu.async_copy(
            a_hbm.at[pl.ds(ebase + c * ECH, ECH)], a_bufs[c], i_sems[c])
    pltpu.sync_copy(b_hbm, b_v)
    pltpu.sync_copy(tab_hbm, tab_v)

    iota = lax.iota(jnp.int32, L)
    cb = b_v[...] + 128
    wcol = lax.shift_right_arithmetic(cb, 1)
    odd = lax.eq(lax.bitwise_and(cb, 1), 1)
    for j in range(256 // L):
        r = j * L + iota
        w = plsc.load_gather(tab_v, [r * 128 + wcol])
        lo = lax.shift_right_arithmetic(lax.shift_left(w, 16), 16)
        hi = lax.shift_right_arithmetic(w, 16)
        col_v[pl.ds(j * L, L)] = lax.select(odd, hi, lo)

    iota2 = iota * 2

    def compute_chunk(a_ref, o_ref):
        @plsc.parallel_loop(0, WCH // L, unroll=8)
        def body(g):
            eb = g * (2 * L)
            av_e = plsc.load_gather(a_ref, [eb + iota2])
            av_o = plsc.load_gather(a_ref, [eb + iota2 + 1])
            ce = plsc.load_gather(col_v, [av_e + 128])
            co = plsc.load_gather(col_v, [av_o + 128])
            word = lax.bitwise_or(
                lax.bitwise_and(ce, 0xFFFF), lax.shift_left(co, 16))
            o_ref[pl.ds(g * L, L)] = word

    out_cp = {}
    for c in range(NCHUNK):
        p = c & 1
        in_cp[c].wait()
        if c >= 2:
            out_cp[c - 2].wait()
        compute_chunk(a_bufs[p], o_bufs[p])
        out_cp[c] = pltpu.async_copy(
            o_bufs[p], out_hbm.at[pl.ds(obase + c * WCH, WCH)], o_sems[p])
        if c + 2 < NCHUNK:
            in_cp[c + 2] = pltpu.async_copy(
                a_hbm.at[pl.ds(ebase + (c + 2) * ECH, ECH)], a_bufs[p],
                i_sems[p])
    out_cp[NCHUNK - 2].wait()
    out_cp[NCHUNK - 1].wait()


def kernel(a, b, table):
    tab_w = lax.bitcast_convert_type(
        table.reshape(TAB_WORDS, 2), jnp.int32)
    a_flat = a.astype(jnp.int32).reshape(N)
    b_vec = jnp.full((L,), b, dtype=jnp.int32)
    out_w = _lut_kernel(tab_w, a_flat, b_vec)
    return lax.bitcast_convert_type(out_w, jnp.int16).reshape(ROWS, COLS)


# speedup vs baseline: 457.9685x; 2.1155x over previous
import functools

import jax
import jax.numpy as jnp
from jax import lax
from jax.experimental import pallas as pl
from jax.experimental.pallas import tpu as pltpu
from jax.experimental.pallas import tpu_sc as plsc

L = 16
NC, NS = 2, 16
NW = NC * NS
ROWS, COLS = 16384, 200
RPT = ROWS // NW
NCHUNK = 8
RCH = RPT // NCHUNK
CSTARTS = list(range(0, 192, 16)) + [184]


@functools.partial(
    pl.kernel,
    out_type=jax.ShapeDtypeStruct((ROWS, COLS), jnp.int16),
    mesh=plsc.VectorSubcoreMesh(core_axis_name="c", subcore_axis_name="s"),
    compiler_params=pltpu.CompilerParams(
        needs_layout_passes=False, use_tc_tiling_on_sc=True),
    scratch_types=[
        pltpu.VMEM((256,), jnp.int32),
        pltpu.VMEM((RCH, COLS), jnp.int32),
        pltpu.VMEM((RCH, COLS), jnp.int32),
        pltpu.VMEM((RCH // 2, COLS), jnp.int32),
        pltpu.VMEM((RCH // 2, COLS), jnp.int32),
        pltpu.SemaphoreType.DMA,
        pltpu.SemaphoreType.DMA,
        pltpu.SemaphoreType.DMA,
        pltpu.SemaphoreType.DMA,
    ],
)
def _lut_kernel(col_hbm, a_hbm, out_hbm,
                col_v, x0_v, x1_v, o0_v, o1_v,
                is0, is1, os0, os1):
    wid = lax.axis_index("s") * NC + lax.axis_index("c")
    rbase = wid * RPT
    x_bufs = (x0_v, x1_v)
    o_bufs = (o0_v, o1_v)
    i_sems = (is0, is1)
    o_sems = (os0, os1)

    in_cp = {}
    for c in range(2):
        in_cp[c] = pltpu.async_copy(
            a_hbm.at[pl.ds(rbase + c * RCH, RCH), :], x_bufs[c], i_sems[c])
    pltpu.sync_copy(col_hbm, col_v)

    iota = lax.iota(jnp.int32, L)

    def compute_chunk(x_ref, o_ref):
        @plsc.parallel_loop(0, RCH // 2, unroll=2)
        def body(q):
            rv_e = jnp.full((L,), 2 * q, jnp.int32)
            rv_o = rv_e + 1
            for c0 in CSTARTS:
                av_e = plsc.load_gather(x_ref, [rv_e, c0 + iota])
                av_o = plsc.load_gather(x_ref, [rv_o, c0 + iota])
                ge = plsc.load_gather(col_v, [av_e + 128])
                go = plsc.load_gather(col_v, [av_o + 128])
                w = lax.bitwise_or(
                    lax.bitwise_and(ge, 0xFFFF), lax.shift_left(go, 16))
                o_ref[q, pl.ds(c0, L)] = w

    out_cp = {}
    for c in range(NCHUNK):
        p = c & 1
        in_cp[c].wait()
        if c >= 2:
            out_cp[c - 2].wait()
        compute_chunk(x_bufs[p], o_bufs[p])
        out_cp[c] = pltpu.async_copy(
            o_bufs[p].bitcast(jnp.int16),
            out_hbm.at[pl.ds(rbase + c * RCH, RCH), :], o_sems[p])
        if c + 2 < NCHUNK:
            in_cp[c + 2] = pltpu.async_copy(
                a_hbm.at[pl.ds(rbase + (c + 2) * RCH, RCH), :], x_bufs[p],
                i_sems[p])
    out_cp[NCHUNK - 2].wait()
    out_cp[NCHUNK - 1].wait()


def kernel(a, b, table):
    idx_b = jnp.asarray(b, jnp.int32) + 128
    column = lax.dynamic_slice_in_dim(table, idx_b, 1, axis=1)
    col_i32 = column.reshape(256).astype(jnp.int32)
    return _lut_kernel(col_i32, a.astype(jnp.int32))


# speedup vs baseline: 517.9437x; 1.1310x over previous
import functools

import jax
import jax.numpy as jnp
from jax import lax
from jax.experimental import pallas as pl
from jax.experimental.pallas import tpu as pltpu
from jax.experimental.pallas import tpu_sc as plsc

L = 16
NC, NS = 2, 16
NW = NC * NS
ROWS, COLS = 16384, 200
RPT = ROWS // NW
NCHUNK = 8
RCH = RPT // NCHUNK
CSTARTS = list(range(0, 192, 16)) + [184]


@functools.partial(
    pl.kernel,
    out_type=jax.ShapeDtypeStruct((ROWS, COLS), jnp.int16),
    mesh=plsc.VectorSubcoreMesh(core_axis_name="c", subcore_axis_name="s"),
    compiler_params=pltpu.CompilerParams(
        needs_layout_passes=False, use_tc_tiling_on_sc=True),
    scratch_types=[
        pltpu.VMEM((256,), jnp.int32),
        pltpu.VMEM((RCH, COLS), jnp.int32),
        pltpu.VMEM((RCH, COLS), jnp.int32),
        pltpu.VMEM((RCH // 2, COLS), jnp.int32),
        pltpu.VMEM((RCH // 2, COLS), jnp.int32),
        pltpu.SemaphoreType.DMA,
        pltpu.SemaphoreType.DMA,
        pltpu.SemaphoreType.DMA,
        pltpu.SemaphoreType.DMA,
    ],
)
def _lut_kernel(col_hbm, a_hbm, out_hbm,
                col_v, x0_v, x1_v, o0_v, o1_v,
                is0, is1, os0, os1):
    wid = lax.axis_index("s") * NC + lax.axis_index("c")
    rbase = wid * RPT
    x_bufs = (x0_v, x1_v)
    o_bufs = (o0_v, o1_v)
    i_sems = (is0, is1)
    o_sems = (os0, os1)

    in_cp = {}
    for c in range(2):
        in_cp[c] = pltpu.async_copy(
            a_hbm.at[pl.ds(rbase + c * RCH, RCH), :], x_bufs[c], i_sems[c])
    pltpu.sync_copy(col_hbm, col_v)

    iota = lax.iota(jnp.int32, L)

    def compute_chunk(x_ref, o_ref):
        @plsc.parallel_loop(0, RCH // 2, unroll=2)
        def body(q):
            for c0 in CSTARTS:
                av_e = x_ref[2 * q, pl.ds(c0, L)]
                av_o = x_ref[2 * q + 1, pl.ds(c0, L)]
                ge = plsc.load_gather(col_v, [av_e + 128])
                go = plsc.load_gather(col_v, [av_o + 128])
                w = lax.bitwise_or(
                    lax.bitwise_and(ge, 0xFFFF), lax.shift_left(go, 16))
                o_ref[q, pl.ds(c0, L)] = w

    out_cp = {}
    for c in range(NCHUNK):
        p = c & 1
        in_cp[c].wait()
        if c >= 2:
            out_cp[c - 2].wait()
        compute_chunk(x_bufs[p], o_bufs[p])
        out_cp[c] = pltpu.async_copy(
            o_bufs[p].bitcast(jnp.int16),
            out_hbm.at[pl.ds(rbase + c * RCH, RCH), :], o_sems[p])
        if c + 2 < NCHUNK:
            in_cp[c + 2] = pltpu.async_copy(
                a_hbm.at[pl.ds(rbase + (c + 2) * RCH, RCH), :], x_bufs[p],
                i_sems[p])
    out_cp[NCHUNK - 2].wait()
    out_cp[NCHUNK - 1].wait()


def kernel(a, b, table):
    idx_b = jnp.asarray(b, jnp.int32) + 128
    column = lax.dynamic_slice_in_dim(table, idx_b, 1, axis=1)
    col_i32 = column.reshape(256).astype(jnp.int32)
    return _lut_kernel(col_i32, a.astype(jnp.int32))
